# baseline (speedup 1.0000x reference)
import jax
import jax.numpy as jnp
from jax import lax
from jax.experimental import pallas as pl
from jax.experimental.pallas import tpu as pltpu
from jax.experimental.pallas import tpu_sc as plsc

NITEM = 1000000
EMB_DIM = 64
BATCH = 16384

NC = 2
NS = 16
LANES = 16
NW = NC * NS
BPW = BATCH // NW
BLK = 128
ICHUNK = 128
NICHUNK = BPW // ICHUNK


def _fetch(t1_hbm, t2_hbm, idx1, idx2, b1, b2, sem):
    base1 = pl.multiple_of((idx1 >> 7) * BLK, BLK)
    base2 = pl.multiple_of((idx2 >> 7) * BLK, BLK)
    pltpu.async_copy(t1_hbm.at[:, pl.ds(base1, BLK)], b1, sem)
    pltpu.async_copy(t2_hbm.at[:, pl.ds(base2, BLK)], b2, sem)


def _sc_body(center_hbm, context_hbm, t1_hbm, t2_hbm, out_hbm,
             cidx_v, xidx_v, c_s, x_s,
             blk0_1, blk0_2, blk1_1, blk1_2, blk2_1, blk2_2, blk3_1, blk3_2,
             blk4_1, blk4_2, blk5_1, blk5_2, blk6_1, blk6_2,
             out_v, sem0, sem1, sem2, sem3, sem4, sem5, sem6):
    wid = lax.axis_index("s") * NC + lax.axis_index("c")
    base = wid * BPW

    for j in range(NICHUNK):
        pltpu.sync_copy(center_hbm.at[pl.ds(base + j * ICHUNK, ICHUNK)],
                        cidx_v.at[j])
        pltpu.sync_copy(context_hbm.at[pl.ds(base + j * ICHUNK, ICHUNK)],
                        xidx_v.at[j])

    def spill_body(g, carry):
        j = g // (ICHUNK // LANES)
        gg = g % (ICHUNK // LANES)
        v1 = cidx_v[j, pl.ds(gg * LANES, LANES)]
        v2 = xidx_v[j, pl.ds(gg * LANES, LANES)]
        for r in range(LANES):
            c_s[g * LANES + r] = v1[r]
            x_s[g * LANES + r] = v2[r]
        return carry

    lax.fori_loop(0, BPW // LANES, spill_body, 0)

    iota = lax.iota(jnp.int32, LANES)
    c_vecs = [kc * LANES + iota for kc in range(EMB_DIM // LANES)]

    slots = [(blk0_1, blk0_2, sem0), (blk1_1, blk1_2, sem1),
             (blk2_1, blk2_2, sem2), (blk3_1, blk3_2, sem3),
             (blk4_1, blk4_2, sem4), (blk5_1, blk5_2, sem5),
             (blk6_1, blk6_2, sem6)]
    NSLOT = len(slots)
    NTURN = (BPW + NSLOT - 1) // NSLOT

    for s, (b1, b2, sem) in enumerate(slots):
        _fetch(t1_hbm, t2_hbm, c_s[s], x_s[s], b1, b2, sem)

    def _dot(b1, b2, col1, col2, acc, lane):
        col1v = jnp.full((LANES,), col1, jnp.int32)
        col2v = jnp.full((LANES,), col2, jnp.int32)
        s = jnp.zeros((LANES,), jnp.float32)
        for kc in range(EMB_DIM // LANES):
            a = plsc.load_gather(b1, [c_vecs[kc], col1v])
            b = plsc.load_gather(b2, [c_vecs[kc], col2v])
            s = s + a * b
        return jnp.where(lane, jnp.sum(s), acc)

    def ring_body(g, acc):
        for s, (b1, b2, sem) in enumerate(slots):
            k = NSLOT * g + s

            def _consume(acc, b1=b1, b2=b2, sem=sem, k=k):
                pltpu.make_async_copy(t1_hbm.at[:, pl.ds(0, BLK)],
                                      b1, sem).wait()
                pltpu.make_async_copy(t1_hbm.at[:, pl.ds(0, BLK)],
                                      b2, sem).wait()
                acc = _dot(b1, b2, c_s[k] & (BLK - 1), x_s[k] & (BLK - 1),
                           acc, iota == (k % LANES))

                @pl.when(k + NSLOT < BPW)
                def _():
                    _fetch(t1_hbm, t2_hbm, c_s[k + NSLOT], x_s[k + NSLOT],
                           b1, b2, sem)

                @pl.when(k % LANES == LANES - 1)
                def _():
                    y = 1.0 / (1.0 + jnp.exp(-acc))
                    out_v[pl.ds((k // LANES) * LANES, LANES)] = y

                return jnp.where(k % LANES == LANES - 1,
                                 jnp.zeros((LANES,), jnp.float32), acc)

            if BPW % NSLOT == 0:
                acc = _consume(acc)
            else:
                acc = jax.lax.cond(k < BPW, _consume, lambda a: a, acc)
        return acc

    lax.fori_loop(0, NTURN, ring_body, jnp.zeros((LANES,), jnp.float32))

    pltpu.sync_copy(out_v, out_hbm.at[pl.ds(base, BPW)])


def kernel(center, context, table1, table2):
    mesh = plsc.VectorSubcoreMesh(core_axis_name="c", subcore_axis_name="s",
                                  num_cores=NC, num_subcores=NS)
    run = pl.kernel(
        _sc_body,
        out_type=jax.ShapeDtypeStruct((BATCH,), jnp.float32),
        mesh=mesh,
        compiler_params=pltpu.CompilerParams(needs_layout_passes=False,
                                             use_tc_tiling_on_sc=True),
        scratch_types=[
            pltpu.VMEM((NICHUNK, ICHUNK), jnp.int32),
            pltpu.VMEM((NICHUNK, ICHUNK), jnp.int32),
            pltpu.SMEM((BPW,), jnp.int32),
            pltpu.SMEM((BPW,), jnp.int32),
            pltpu.VMEM((EMB_DIM, BLK), jnp.float32),
            pltpu.VMEM((EMB_DIM, BLK), jnp.float32),
            pltpu.VMEM((EMB_DIM, BLK), jnp.float32),
            pltpu.VMEM((EMB_DIM, BLK), jnp.float32),
            pltpu.VMEM((EMB_DIM, BLK), jnp.float32),
            pltpu.VMEM((EMB_DIM, BLK), jnp.float32),
            pltpu.VMEM((EMB_DIM, BLK), jnp.float32),
            pltpu.VMEM((EMB_DIM, BLK), jnp.float32),
            pltpu.VMEM((EMB_DIM, BLK), jnp.float32),
            pltpu.VMEM((EMB_DIM, BLK), jnp.float32),
            pltpu.VMEM((EMB_DIM, BLK), jnp.float32),
            pltpu.VMEM((EMB_DIM, BLK), jnp.float32),
            pltpu.VMEM((EMB_DIM, BLK), jnp.float32),
            pltpu.VMEM((EMB_DIM, BLK), jnp.float32),
            pltpu.VMEM((BPW,), jnp.float32),
            pltpu.SemaphoreType.DMA,
            pltpu.SemaphoreType.DMA,
            pltpu.SemaphoreType.DMA,
            pltpu.SemaphoreType.DMA,
            pltpu.SemaphoreType.DMA,
            pltpu.SemaphoreType.DMA,
            pltpu.SemaphoreType.DMA,
        ],
    )
    return run(center.astype(jnp.int32), context.astype(jnp.int32),
               jnp.swapaxes(table1, 0, 1), jnp.swapaxes(table2, 0, 1))
